# baseline (speedup 1.0000x reference)
import functools

import jax
import jax.numpy as jnp
from jax import lax
from jax.experimental import pallas as pl
from jax.experimental.pallas import tpu as pltpu
from jax.experimental.pallas import tpu_sc as plsc

_B = 64
_ROW = 576 * 768
_NW = 32
_CH = _ROW // _NW
_NBUF = 4
_PF = 2
_LANES = 16


def _sc_body(patch_hbm, pos_hbm, out_hbm, pos_v, bufs, in_sems, out_sems):
    wid = lax.axis_index("s") * 2 + lax.axis_index("c")
    band = wid * _CH

    pltpu.sync_copy(pos_hbm.at[pl.ds(band, _CH)], pos_v)

    def in_copy(i, k):
        return pltpu.make_async_copy(
            patch_hbm.at[pl.ds(i * _ROW + band, _CH)], bufs.at[k], in_sems.at[k]
        )

    def out_copy(i, k):
        return pltpu.make_async_copy(
            bufs.at[k], out_hbm.at[pl.ds(i * _ROW + band, _CH)], out_sems.at[k]
        )

    def add_band(k):
        def body(j, c):
            s = j * _LANES
            bufs[k, pl.ds(s, _LANES)] = (
                bufs[k, pl.ds(s, _LANES)] + pos_v[pl.ds(s, _LANES)]
            )
            return c

        lax.fori_loop(0, _CH // _LANES, body, 0)

    def step(i, k, *, prefetch=True, wait_out=True):
        in_copy(i, k).wait()
        add_band(k)
        out_copy(i, k).start()
        if prefetch:
            j = i + _PF
            kj = (k + _PF) % _NBUF
            if wait_out:
                out_copy(j - _NBUF, kj).wait()
            in_copy(j, kj).start()

    for k in range(_PF):
        in_copy(k, k).start()

    for k in range(_NBUF):
        step(k, k, wait_out=(k + _PF >= _NBUF))

    def group(g, c):
        base = g * _NBUF
        for k in range(_NBUF):
            step(base + k, k)
        return c

    lax.fori_loop(1, _B // _NBUF - 1, group, 0)

    base = _B - _NBUF
    for k in range(_NBUF):
        step(base + k, k, prefetch=(k + _PF < _NBUF))

    for k in range(_NBUF):
        out_copy(_B - _NBUF + k, k).wait()


def kernel(patch, pos_table):
    B, N, D = patch.shape
    sc_add = pl.kernel(
        _sc_body,
        out_type=jax.ShapeDtypeStruct((B * N * D,), patch.dtype),
        mesh=plsc.VectorSubcoreMesh(core_axis_name="c", subcore_axis_name="s"),
        scratch_types=[
            pltpu.VMEM((_CH,), patch.dtype),
            pltpu.VMEM((_NBUF, _CH), patch.dtype),
            pltpu.SemaphoreType.DMA((_NBUF,)),
            pltpu.SemaphoreType.DMA((_NBUF,)),
        ],
    )
    out = sc_add(patch.reshape(-1), pos_table.reshape(-1))
    return out.reshape(B, N, D)


# speedup vs baseline: 8.8418x; 8.8418x over previous
import functools

import jax
import jax.numpy as jnp
from jax import lax
from jax.experimental import pallas as pl
from jax.experimental.pallas import tpu as pltpu

_CB = 4
_NBUF = 8
_PF = 3


def _in_copy(patch_hbm, bufs, in_sems, i, k):
    return pltpu.make_async_copy(
        patch_hbm.at[pl.ds(i * _CB, _CB)], bufs.at[k], in_sems.at[k]
    )


def _out_copy(out_hbm, bufs, out_sems, i, k):
    return pltpu.make_async_copy(
        bufs.at[k], out_hbm.at[pl.ds(i * _CB, _CB)], out_sems.at[k]
    )


def _pipe_kernel(nch, patch_hbm, pos_ref, out_hbm, bufs, in_sems, out_sems):
    pos = pos_ref[...][None]

    def step(i, k, *, prefetch=True, wait_out=True):
        _in_copy(patch_hbm, bufs, in_sems, i, k).wait()
        bufs[k] = bufs[k] + pos
        _out_copy(out_hbm, bufs, out_sems, i, k).start()
        if prefetch:
            j = i + _PF
            kj = (k + _PF) % _NBUF
            if wait_out:
                _out_copy(out_hbm, bufs, out_sems, j - _NBUF, kj).wait()
            _in_copy(patch_hbm, bufs, in_sems, j, kj).start()

    for k in range(_PF):
        _in_copy(patch_hbm, bufs, in_sems, k, k).start()

    for k in range(_NBUF):
        step(k, k, wait_out=(k + _PF >= _NBUF))

    def body(g, carry):
        base = g * _NBUF
        for k in range(_NBUF):
            step(base + k, k)
        return carry

    lax.fori_loop(1, nch // _NBUF - 1, body, 0)

    base = nch - _NBUF
    for k in range(_NBUF):
        step(base + k, k, prefetch=(k + _PF < _NBUF))

    for k in range(_NBUF):
        _out_copy(out_hbm, bufs, out_sems, nch - _NBUF + k, k).wait()


def kernel(patch, pos_table):
    B, N, D = patch.shape
    nch = B // _CB
    return pl.pallas_call(
        functools.partial(_pipe_kernel, nch),
        in_specs=[
            pl.BlockSpec(memory_space=pl.ANY),
            pl.BlockSpec(memory_space=pltpu.VMEM),
        ],
        out_specs=pl.BlockSpec(memory_space=pl.ANY),
        out_shape=jax.ShapeDtypeStruct((B, N, D), patch.dtype),
        scratch_shapes=[
            pltpu.VMEM((_NBUF, _CB, N, D), patch.dtype),
            pltpu.SemaphoreType.DMA((_NBUF,)),
            pltpu.SemaphoreType.DMA((_NBUF,)),
        ],
    )(patch, pos_table)


# speedup vs baseline: 9.0485x; 1.0234x over previous
import jax
import jax.numpy as jnp
from jax.experimental import pallas as pl
from jax.experimental.pallas import tpu as pltpu


def _add_kernel(patch_ref, pos_ref, out_ref):
    out_ref[...] = patch_ref[...] + pos_ref[...][None]


_BB = 8


def kernel(patch, pos_table):
    B, N, D = patch.shape
    return pl.pallas_call(
        _add_kernel,
        grid=(B // _BB,),
        in_specs=[
            pl.BlockSpec((_BB, N, D), lambda b: (b, 0, 0)),
            pl.BlockSpec(memory_space=pltpu.VMEM),
        ],
        out_specs=pl.BlockSpec((_BB, N, D), lambda b: (b, 0, 0)),
        out_shape=jax.ShapeDtypeStruct((B, N, D), patch.dtype),
    )(patch, pos_table)
